# baseline (speedup 1.0000x reference)
import functools

import jax
import jax.numpy as jnp
from jax import lax
from jax.experimental import pallas as pl
from jax.experimental.pallas import tpu as pltpu
from jax.experimental.pallas import tpu_sc as plsc

_DIM = 32
_G = 128
_S = 8
_NC = 2
_NS = 16
_NW = _NC * _NS


@functools.partial(jax.jit, static_argnums=(2,))
def _gather_rows(idx2d, table, n_rows):
    rows_per_w = n_rows // _NW
    chunks = rows_per_w // _S
    mesh = plsc.VectorSubcoreMesh(core_axis_name="c", subcore_axis_name="s")

    @functools.partial(
        pl.kernel,
        mesh=mesh,
        out_type=jax.ShapeDtypeStruct((n_rows * _G, _DIM), jnp.float32),
        scratch_types=[
            pltpu.VMEM((_S, _G), jnp.int32),
            pltpu.VMEM((_S * _G, _DIM), jnp.float32),
            pltpu.SemaphoreType.DMA,
        ],
        compiler_params=pltpu.CompilerParams(use_tc_tiling_on_sc=False),
    )
    def gather(idx_hbm, table_hbm, out_hbm, idx_v, rows_v, sem):
        wid = lax.axis_index("s") * _NC + lax.axis_index("c")
        row0 = wid * rows_per_w

        def body(i, carry):
            r = row0 + i * _S
            pltpu.sync_copy(idx_hbm.at[pl.ds(r, _S)], idx_v)
            copies = [
                pltpu.async_copy(
                    table_hbm.at[idx_v.at[j]],
                    rows_v.at[pl.ds(j * _G, _G)],
                    sem,
                )
                for j in range(_S)
            ]
            for c in copies:
                c.wait()
            pltpu.sync_copy(rows_v, out_hbm.at[pl.ds(r * _G, _S * _G)])
            return carry

        lax.fori_loop(0, chunks, body, 0)

    return gather(idx2d, table)


def _transpose_out(g128, l, b):
    bc = 2048
    nb = b // bc

    def body(in_ref, out_ref):
        x = in_ref[...]
        x = x.reshape(bc // 4, 4, _DIM)
        x = jnp.transpose(x, (2, 0, 1))
        out_ref[0] = x.reshape(_DIM, bc)

    return pl.pallas_call(
        body,
        grid=(l, nb),
        in_specs=[pl.BlockSpec((bc // 4, 128), lambda i, c: (i * nb + c, 0))],
        out_specs=pl.BlockSpec((1, _DIM, bc), lambda i, c: (i, 0, c)),
        out_shape=jax.ShapeDtypeStruct((l, _DIM, b), jnp.float32),
    )(g128)


def kernel(indices, table):
    b, l = indices.shape
    n = b * l
    idx2d = jnp.transpose(indices).astype(jnp.int32).reshape(n // _G, _G)
    out = _gather_rows(idx2d, table, n // _G)
    g128 = out.reshape(n * _DIM // 128, 128)
    out3d = _transpose_out(g128, l, b)
    return jnp.transpose(out3d, (2, 0, 1))


# speedup vs baseline: 4.0348x; 4.0348x over previous
import functools

import jax
import jax.numpy as jnp
from jax import lax
from jax.experimental import pallas as pl
from jax.experimental.pallas import tpu as pltpu
from jax.experimental.pallas import tpu_sc as plsc

_DIM = 32
_G = 128
_CHUNK = 2048
_NC = 2
_NS = 16
_NW = _NC * _NS
_CB = 2048
_BC = 2048


def _prep_table(table_t, v):
    q4 = _CB // 4
    grid = (v + _CB - 1) // _CB

    def body(in_ref, out_ref):
        x = in_ref[...]
        xt = jnp.transpose(x)
        for q in range(4):
            out_ref[:, 32 * q:32 * (q + 1)] = xt[q * q4:(q + 1) * q4, :]

    return pl.pallas_call(
        body,
        grid=(grid,),
        in_specs=[pl.BlockSpec((_DIM, _CB), lambda c: (0, c))],
        out_specs=pl.BlockSpec((q4, 128), lambda c: (c, 0)),
        out_shape=jax.ShapeDtypeStruct((grid * q4, 128), jnp.float32),
    )(table_t)


def _transpose_out(g128, l, b):
    q4 = _BC // 4
    nb = b // _BC

    def body(in_ref, out_ref):
        for q in range(4):
            yq = in_ref[:, 32 * q:32 * (q + 1)]
            out_ref[0, :, q4 * q:q4 * (q + 1)] = jnp.transpose(yq)

    return pl.pallas_call(
        body,
        grid=(l, nb),
        in_specs=[pl.BlockSpec((q4, 128), lambda i, c: (i * nb + c, 0))],
        out_specs=pl.BlockSpec((1, _DIM, _BC), lambda i, c: (i, 0, c)),
        out_shape=jax.ShapeDtypeStruct((l, _DIM, b), jnp.float32),
    )(g128)


@functools.partial(jax.jit, static_argnums=(2,))
def _gather_rows(idx2d, table, n_rows):
    rows_per_chunk = _CHUNK // _G
    chunks = n_rows // rows_per_chunk
    chunks_per_w = chunks // _NW
    n_lines = n_rows * _G // 4
    mesh = plsc.VectorSubcoreMesh(core_axis_name="c", subcore_axis_name="s")

    @functools.partial(
        pl.kernel,
        mesh=mesh,
        out_type=jax.ShapeDtypeStruct((n_lines, 128), jnp.float32),
        scratch_types=[
            pltpu.VMEM((rows_per_chunk, _G), jnp.int32),
            pltpu.VMEM((_CHUNK, _DIM), jnp.float32),
            pltpu.SemaphoreType.DMA,
        ],
        compiler_params=pltpu.CompilerParams(use_tc_tiling_on_sc=False),
    )
    def gather(idx_hbm, table_hbm, out_hbm, idx_v, rows_v, sem):
        wid = lax.axis_index("s") * _NC + lax.axis_index("c")
        c0 = wid * chunks_per_w

        def body(i, carry):
            c = c0 + i
            pltpu.sync_copy(idx_hbm.at[pl.ds(c * rows_per_chunk, rows_per_chunk)], idx_v)
            copies = [
                pltpu.async_copy(
                    table_hbm.at[idx_v.at[j]],
                    rows_v.at[pl.ds(j * _G, _G)],
                    sem,
                )
                for j in range(rows_per_chunk)
            ]
            for cp in copies:
                cp.wait()
            for q in range(4):
                pltpu.sync_copy(
                    rows_v.at[pl.ds(512 * q, 512)],
                    out_hbm.at[pl.ds(c * 512, 512), pl.ds(32 * q, 32)],
                )
            return carry

        lax.fori_loop(0, chunks_per_w, body, 0)

    return gather(idx2d, table)


def kernel(indices, table):
    b, l = indices.shape
    n = b * l
    v = table.shape[0]
    vp = ((v + _CB - 1) // _CB) * _CB
    table_lin = _prep_table(jnp.transpose(table), v).reshape(vp, _DIM)
    idx = jnp.transpose(indices).astype(jnp.int32)
    jp = (idx & ~2047) + ((idx & 511) << 2) + ((idx & 2047) >> 9)
    idx2d = jp.reshape(n // _G, _G)
    g128 = _gather_rows(idx2d, table_lin, n // _G)
    out3d = _transpose_out(g128, l, b)
    return jnp.transpose(out3d, (2, 0, 1))


# speedup vs baseline: 10.2750x; 2.5466x over previous
import functools

import jax
import jax.numpy as jnp
from jax import lax
from jax.experimental import pallas as pl
from jax.experimental.pallas import tpu as pltpu
from jax.experimental.pallas import tpu_sc as plsc

_DIM = 32
_G = 128
_CHUNK = 2048
_NC = 2
_NS = 16
_NW = _NC * _NS
_CB = 2048
_BC = 2048


def _prep_table(table_t, v):
    q4 = _CB // 4
    grid = (v + _CB - 1) // _CB

    def body(in_ref, out_ref):
        x = in_ref[...]
        y = jnp.concatenate([x[:, q * q4:(q + 1) * q4] for q in range(4)], axis=0)
        out_ref[...] = jnp.transpose(y)

    return pl.pallas_call(
        body,
        grid=(grid,),
        in_specs=[pl.BlockSpec((_DIM, _CB), lambda c: (0, c))],
        out_specs=pl.BlockSpec((q4, 128), lambda c: (c, 0)),
        out_shape=jax.ShapeDtypeStruct((grid * q4, 128), jnp.float32),
    )(table_t)


def _transpose_out(g128, l, b):
    q4 = _BC // 4
    nb = b // _BC

    def body(in_ref, out_ref):
        xt = jnp.transpose(in_ref[...])
        for q in range(4):
            out_ref[0, :, q4 * q:q4 * (q + 1)] = xt[32 * q:32 * (q + 1), :]

    return pl.pallas_call(
        body,
        grid=(l, nb),
        in_specs=[pl.BlockSpec((q4, 128), lambda i, c: (i * nb + c, 0))],
        out_specs=pl.BlockSpec((1, _DIM, _BC), lambda i, c: (i, 0, c)),
        out_shape=jax.ShapeDtypeStruct((l, _DIM, b), jnp.float32),
    )(g128)


@functools.partial(jax.jit, static_argnums=(2,))
def _gather_rows(idx2d, table, n_rows):
    rows_per_chunk = _CHUNK // _G
    chunks = n_rows // rows_per_chunk
    chunks_per_w = chunks // _NW
    n_lines = n_rows * _G // 4
    mesh = plsc.VectorSubcoreMesh(core_axis_name="c", subcore_axis_name="s")

    @functools.partial(
        pl.kernel,
        mesh=mesh,
        out_type=jax.ShapeDtypeStruct((n_lines, 128), jnp.float32),
        scratch_types=[
            pltpu.VMEM((rows_per_chunk, _G), jnp.int32),
            pltpu.VMEM((_CHUNK, _DIM), jnp.float32),
            pltpu.SemaphoreType.DMA,
        ],
        compiler_params=pltpu.CompilerParams(use_tc_tiling_on_sc=False),
    )
    def gather(idx_hbm, table_hbm, out_hbm, idx_v, rows_v, sem):
        wid = lax.axis_index("s") * _NC + lax.axis_index("c")
        c0 = wid * chunks_per_w

        def body(i, carry):
            c = c0 + i
            pltpu.sync_copy(idx_hbm.at[pl.ds(c * rows_per_chunk, rows_per_chunk)], idx_v)
            copies = [
                pltpu.async_copy(
                    table_hbm.at[idx_v.at[j]],
                    rows_v.at[pl.ds(j * _G, _G)],
                    sem,
                )
                for j in range(rows_per_chunk)
            ]
            for cp in copies:
                cp.wait()
            for q in range(4):
                pltpu.sync_copy(
                    rows_v.at[pl.ds(512 * q, 512)],
                    out_hbm.at[pl.ds(c * 512, 512), pl.ds(32 * q, 32)],
                )
            return carry

        lax.fori_loop(0, chunks_per_w, body, 0)

    return gather(idx2d, table)


def kernel(indices, table):
    b, l = indices.shape
    n = b * l
    v = table.shape[0]
    vp = ((v + _CB - 1) // _CB) * _CB
    table_lin = _prep_table(jnp.transpose(table), v).reshape(vp, _DIM)
    idx = jnp.transpose(indices).astype(jnp.int32)
    jp = (idx & ~2047) + ((idx & 511) << 2) + ((idx & 2047) >> 9)
    idx2d = jp.reshape(n // _G, _G)
    g128 = _gather_rows(idx2d, table_lin, n // _G)
    out3d = _transpose_out(g128, l, b)
    return jnp.transpose(out3d, (2, 0, 1))


# speedup vs baseline: 11.0232x; 1.0728x over previous
import functools

import jax
import jax.numpy as jnp
from jax import lax
from jax.experimental import pallas as pl
from jax.experimental.pallas import tpu as pltpu
from jax.experimental.pallas import tpu_sc as plsc

_DIM = 32
_G = 128
_CHUNK = 2048
_NC = 2
_NS = 16
_NW = _NC * _NS
_CB = 2048
_PB = 8192
_BC = 16384
_P = 5


def _prep_table(table_t, v):
    q4 = _CB // 4
    sb = _PB // _CB
    grid = (v + _PB - 1) // _PB

    def body(in_ref, out_ref):
        x = in_ref[...]
        rows = [
            jnp.concatenate(
                [x[:, s * _CB + q * q4:s * _CB + (q + 1) * q4] for s in range(sb)],
                axis=1,
            )
            for q in range(4)
        ]
        y = jnp.concatenate(rows, axis=0)
        out_ref[...] = jnp.transpose(y)

    return pl.pallas_call(
        body,
        grid=(grid,),
        in_specs=[pl.BlockSpec((_DIM, _PB), lambda c: (0, c))],
        out_specs=pl.BlockSpec((_PB // 4, 128), lambda c: (c, 0)),
        out_shape=jax.ShapeDtypeStruct((grid * _PB // 4, 128), jnp.float32),
    )(table_t)


def _transpose_piece(g128_p, acc, piece, l_pp, l, b):
    sb = _BC // _CHUNK

    def body(*refs):
        in_ref, out_ref = refs[0], refs[-1]
        xt = jnp.transpose(in_ref[...])
        for c in range(sb):
            for q in range(4):
                out_ref[0, :, c * _CHUNK + 512 * q:c * _CHUNK + 512 * (q + 1)] = (
                    xt[32 * q:32 * (q + 1), c * 512:(c + 1) * 512])

    in_specs = [pl.BlockSpec((_BC // 4, 128), lambda i: (i, 0))]
    operands = [g128_p]
    kwargs = {}
    if acc is not None:
        in_specs.append(pl.BlockSpec(memory_space=pl.ANY))
        operands.append(acc)
        kwargs["input_output_aliases"] = {1: 0}

    return pl.pallas_call(
        body,
        grid=(l_pp,),
        in_specs=in_specs,
        out_specs=pl.BlockSpec((1, _DIM, _BC), lambda i, piece=piece: (piece * l_pp + i, 0, 0)),
        out_shape=jax.ShapeDtypeStruct((l, _DIM, b), jnp.float32),
        **kwargs,
    )(*operands)


@functools.partial(jax.jit, static_argnums=(2, 3, 4))
def _gather_rows(idx2d, table, n_rows, piece, n_pieces):
    rows_per_chunk = _CHUNK // _G
    chunks = n_rows // rows_per_chunk
    chunks_pp = chunks // n_pieces
    chunks_per_w = chunks_pp // _NW
    n_lines = chunks_pp * 512
    mesh = plsc.VectorSubcoreMesh(core_axis_name="c", subcore_axis_name="s")

    @functools.partial(
        pl.kernel,
        mesh=mesh,
        out_type=jax.ShapeDtypeStruct((n_lines, 128), jnp.float32),
        scratch_types=[
            pltpu.VMEM((rows_per_chunk, _G), jnp.int32),
            pltpu.VMEM((_CHUNK, _DIM), jnp.float32),
            pltpu.SemaphoreType.DMA,
        ],
        compiler_params=pltpu.CompilerParams(use_tc_tiling_on_sc=False),
    )
    def gather(idx_hbm, table_hbm, out_hbm, idx_v, rows_v, sem):
        wid = lax.axis_index("s") * _NC + lax.axis_index("c")
        c0 = piece * chunks_pp + wid * chunks_per_w

        def body(i, carry):
            c = c0 + i
            co = (wid * chunks_per_w + i) * 512
            pltpu.sync_copy(idx_hbm.at[pl.ds(c * rows_per_chunk, rows_per_chunk)], idx_v)
            copies = [
                pltpu.async_copy(
                    table_hbm.at[idx_v.at[j]],
                    rows_v.at[pl.ds(j * _G, _G)],
                    sem,
                )
                for j in range(rows_per_chunk)
            ]
            for cp in copies:
                cp.wait()
            for q in range(4):
                pltpu.sync_copy(
                    rows_v.at[pl.ds(512 * q, 512)],
                    out_hbm.at[pl.ds(co, 512), pl.ds(32 * q, 32)],
                )
            return carry

        lax.fori_loop(0, chunks_per_w, body, 0)

    return gather(idx2d, table)


def kernel(indices, table):
    b, l = indices.shape
    n = b * l
    v = table.shape[0]
    vp = ((v + _PB - 1) // _PB) * _PB
    table_lin = _prep_table(jnp.transpose(table), v).reshape(vp, _DIM)
    idx = jnp.transpose(indices).astype(jnp.int32)
    jp = (idx & ~2047) + ((idx & 511) << 2) + ((idx & 2047) >> 9)
    idx2d = jp.reshape(n // _G, _G)
    l_pp = l // _P
    acc = None
    for p in range(_P):
        g128_p = _gather_rows(idx2d, table_lin, n // _G, p, _P)
        acc = _transpose_piece(g128_p, acc, p, l_pp, l, b)
    return jnp.transpose(acc, (2, 0, 1))
